# baseline (speedup 1.0000x reference)
import jax
import jax.numpy as jnp
import numpy as np
from jax.experimental import pallas as pl
from jax.experimental.pallas import tpu as pltpu

NUM_THETAS = 16
NUM_PHIS = 16
BUMP_STEPS = 16
RADIUS = 1.1
N_GRAPHS = 8
D = NUM_THETAS * NUM_PHIS
SD = BUMP_STEPS * D

BLK_N = 8192

_LIN = np.linspace(-RADIUS, RADIUS, BUMP_STEPS).astype(np.float32)


def _directions():
    theta = jnp.linspace(0.0, jnp.pi, NUM_THETAS)
    phi = jnp.linspace(0.0, 2.0 * jnp.pi, NUM_PHIS)
    mt, mp = jnp.meshgrid(theta, phi, indexing="ij")
    v = jnp.stack(
        [
            (jnp.sin(mt) * jnp.cos(mp)).reshape(-1),
            (jnp.sin(mt) * jnp.sin(mp)).reshape(-1),
            jnp.cos(mt).reshape(-1),
        ],
        axis=0,
    )
    return v.astype(jnp.float32)


def _fused_kernel(xt_ref, batch_ref, v_ref, out_ref):
    @pl.when(pl.program_id(0) == 0)
    def _init():
        out_ref[...] = jnp.zeros_like(out_ref)

    nh100 = jax.lax.dot_general(
        xt_ref[...], v_ref[...], (((0,), (0,)), ((), ())),
        preferred_element_type=jnp.float32,
    )

    parts = []
    for s in range(BUMP_STEPS):
        arg = float(100.0 * _LIN[s]) - nh100
        parts.append(jnp.tanh(arg))
    tanh_all = jnp.concatenate(parts, axis=1)

    b_ids = jax.lax.broadcasted_iota(jnp.int32, (N_GRAPHS, BLK_N), 0)
    onehot = (b_ids == batch_ref[0]).astype(jnp.float32)
    seg = jax.lax.dot_general(
        onehot, tanh_all, (((1,), (0,)), ((), ())),
        preferred_element_type=jnp.float32,
    )
    count = jnp.sum(onehot, axis=1, keepdims=True)
    out_ref[...] += 0.5 * seg + 0.5 * count


def kernel(x, batch):
    n = x.shape[0]
    xt = jnp.zeros((8, n), dtype=jnp.float32).at[:3, :].set(x.T)
    v = jnp.zeros((8, D), dtype=jnp.float32).at[:3, :].set(100.0 * _directions())
    nblk = n // BLK_N
    batch3 = batch.reshape(nblk, 1, BLK_N)

    out = pl.pallas_call(
        _fused_kernel,
        grid=(nblk,),
        in_specs=[
            pl.BlockSpec((8, BLK_N), lambda g: (0, g)),
            pl.BlockSpec((1, 1, BLK_N), lambda g: (g, 0, 0)),
            pl.BlockSpec((8, D), lambda g: (0, 0)),
        ],
        out_specs=pl.BlockSpec((N_GRAPHS, SD), lambda g: (0, 0)),
        out_shape=jax.ShapeDtypeStruct((N_GRAPHS, SD), jnp.float32),
        compiler_params=pltpu.CompilerParams(
            vmem_limit_bytes=100 * 1024 * 1024),
    )(xt, batch3, v)

    return out.reshape(N_GRAPHS, BUMP_STEPS, NUM_THETAS, NUM_PHIS)
